# baseline (device time: 30684 ns/iter reference)
import jax
import jax.numpy as jnp
from jax import lax
from jax.experimental import pallas as pl
from jax.experimental.pallas import tpu as pltpu

N_Y = 4


def kernel(partial, resid, gamma):
    m, d = resid.shape
    gamma2 = gamma.reshape(1, d)

    def body(partial_ref, resid_ref, gamma_ref, out_ref,
             sendbuf, comm, send_sems, recv_sems):
        my_x = lax.axis_index("x")
        my_y = lax.axis_index("y")
        my_z = lax.axis_index("z")

        barrier_sem = pltpu.get_barrier_semaphore()
        for dy in (1, 2, 3):
            tgt = lax.rem(my_y + dy, N_Y)
            pl.semaphore_signal(
                barrier_sem, inc=1,
                device_id=(my_x, tgt, my_z),
                device_id_type=pl.DeviceIdType.MESH,
            )
        pl.semaphore_wait(barrier_sem, 3)

        sendbuf[...] = partial_ref[0].astype(jnp.bfloat16)

        rdmas = []
        for dy in (1, 2, 3):
            tgt = lax.rem(my_y + dy, N_Y)
            rdma = pltpu.make_async_remote_copy(
                src_ref=sendbuf,
                dst_ref=comm.at[dy - 1],
                send_sem=send_sems.at[dy - 1],
                recv_sem=recv_sems.at[dy - 1],
                device_id=(my_x, tgt, my_z),
                device_id_type=pl.DeviceIdType.MESH,
            )
            rdma.start()
            rdmas.append(rdma)

        for r in rdmas:
            r.wait_recv()

        acc = (partial_ref[0]
               + comm[0].astype(jnp.float32)
               + comm[1].astype(jnp.float32)
               + comm[2].astype(jnp.float32))
        y = acc + resid_ref[...]
        rms = jnp.sqrt(jnp.mean(y * y, axis=-1, keepdims=True) + 1e-6)
        out_ref[...] = y / rms * gamma_ref[...]

        for r in rdmas:
            r.wait_send()

    return pl.pallas_call(
        body,
        out_shape=jax.ShapeDtypeStruct((m, d), jnp.float32),
        in_specs=[
            pl.BlockSpec(memory_space=pltpu.VMEM),
            pl.BlockSpec(memory_space=pltpu.VMEM),
            pl.BlockSpec(memory_space=pltpu.VMEM),
        ],
        out_specs=pl.BlockSpec(memory_space=pltpu.VMEM),
        scratch_shapes=[
            pltpu.VMEM((m, d), jnp.bfloat16),
            pltpu.VMEM((3, m, d), jnp.bfloat16),
            pltpu.SemaphoreType.DMA((3,)),
            pltpu.SemaphoreType.DMA((3,)),
        ],
        compiler_params=pltpu.CompilerParams(collective_id=0),
    )(partial, resid, gamma2)


# device time: 19568 ns/iter; 1.5681x vs baseline; 1.5681x over previous
import jax
import jax.numpy as jnp
from jax import lax
from jax.experimental import pallas as pl
from jax.experimental.pallas import tpu as pltpu

N_Y = 4


def kernel(partial, resid, gamma):
    m, d = resid.shape
    qm = m // N_Y
    gamma2 = gamma.reshape(1, d)

    def body(partial_ref, resid_ref, gamma_ref, out_ref,
             sendbuf, rs_buf, ag_buf,
             rs_send, rs_recv, ag_send, ag_recv):
        my_x = lax.axis_index("x")
        my_y = lax.axis_index("y")
        my_z = lax.axis_index("z")

        barrier_sem = pltpu.get_barrier_semaphore()
        for dy in (1, 2, 3):
            tgt = lax.rem(my_y + dy, N_Y)
            pl.semaphore_signal(
                barrier_sem, inc=1,
                device_id=(my_x, tgt, my_z),
                device_id_type=pl.DeviceIdType.MESH,
            )
        pl.semaphore_wait(barrier_sem, 3)

        sendbuf[...] = partial_ref[0].astype(jnp.bfloat16)

        rs_rdmas = []
        for dy in (1, 2, 3):
            tgt = lax.rem(my_y + dy, N_Y)
            rdma = pltpu.make_async_remote_copy(
                src_ref=sendbuf.at[pl.ds(tgt * qm, qm)],
                dst_ref=rs_buf.at[dy - 1],
                send_sem=rs_send.at[dy - 1],
                recv_sem=rs_recv.at[dy - 1],
                device_id=(my_x, tgt, my_z),
                device_id_type=pl.DeviceIdType.MESH,
            )
            rdma.start()
            rs_rdmas.append(rdma)

        for r in rs_rdmas:
            r.wait_recv()

        row0 = my_y * qm
        q_own = partial_ref[0, pl.ds(row0, qm), :]
        acc = (q_own
               + rs_buf[0].astype(jnp.float32)
               + rs_buf[1].astype(jnp.float32)
               + rs_buf[2].astype(jnp.float32))
        y = acc + resid_ref[pl.ds(row0, qm), :]
        rms = jnp.sqrt(jnp.mean(y * y, axis=-1, keepdims=True) + 1e-6)
        out_q = y / rms * gamma_ref[...]

        ag_buf[3] = out_q.astype(jnp.bfloat16)
        ag_rdmas = []
        for dy in (1, 2, 3):
            tgt = lax.rem(my_y + dy, N_Y)
            rdma = pltpu.make_async_remote_copy(
                src_ref=ag_buf.at[3],
                dst_ref=ag_buf.at[dy - 1],
                send_sem=ag_send.at[dy - 1],
                recv_sem=ag_recv.at[dy - 1],
                device_id=(my_x, tgt, my_z),
                device_id_type=pl.DeviceIdType.MESH,
            )
            rdma.start()
            ag_rdmas.append(rdma)

        out_ref[pl.ds(row0, qm), :] = out_q
        for r in rs_rdmas:
            r.wait_send()

        for s in range(3):
            ag_rdmas[s].wait_recv()
            src_y = lax.rem(my_y - (s + 1) + N_Y, N_Y)
            out_ref[pl.ds(src_y * qm, qm), :] = ag_buf[s].astype(jnp.float32)
        for r in ag_rdmas:
            r.wait_send()

    return pl.pallas_call(
        body,
        out_shape=jax.ShapeDtypeStruct((m, d), jnp.float32),
        in_specs=[
            pl.BlockSpec(memory_space=pltpu.VMEM),
            pl.BlockSpec(memory_space=pltpu.VMEM),
            pl.BlockSpec(memory_space=pltpu.VMEM),
        ],
        out_specs=pl.BlockSpec(memory_space=pltpu.VMEM),
        scratch_shapes=[
            pltpu.VMEM((m, d), jnp.bfloat16),
            pltpu.VMEM((3, qm, d), jnp.bfloat16),
            pltpu.VMEM((4, qm, d), jnp.bfloat16),
            pltpu.SemaphoreType.DMA((3,)),
            pltpu.SemaphoreType.DMA((3,)),
            pltpu.SemaphoreType.DMA((3,)),
            pltpu.SemaphoreType.DMA((3,)),
        ],
        compiler_params=pltpu.CompilerParams(collective_id=0),
    )(partial, resid, gamma2)
